# baseline (speedup 1.0000x reference)
import functools
import numpy as np
import jax
import jax.numpy as jnp
from jax import lax
from jax.experimental import pallas as pl
from jax.experimental.pallas import tpu as pltpu
from jax.experimental.pallas import tpu_sc as plsc

_ELT = 64
_ROW = 65536
_T2 = np.float32((2.0 * np.sin(np.radians(10.0) / 2.0)) ** 2)

_TCS = 48
_SCS = _ELT - _TCS

_C = 1024
_R = _ELT * _ROW // _C
_RSEG = _ROW // _C

_NC, _NS, _L = 2, 16, 16
_NW = _NC * _NS
_WPS = _NW // _SCS
_NROWS = _ELT * _ROW // 128
_ROWSEG = _ROW // 128
_WROWS = _ROWSEG // _WPS
_PR = 64
_NCHUNK = _WROWS // _PR

_LN2 = np.float32(0.6931471805599453)
_SQRT2 = np.float32(1.4142135623730951)


def _tc_body(p_ref, o_ref, h_ref, lam_ref, out_ref):
    d = p_ref[...] - o_ref[...]
    d2 = d * d
    s2 = d2[0] + d2[1] + d2[2]
    h = h_ref[0, 0, 0]
    lam = lam_ref[0, 0, 0]
    p = h * jnp.exp(s2 * (-1.0 / _T2) * lam) + (1.0 - h)
    log_p = jnp.where(s2 < _T2, jnp.log(p), 0.0)
    out_ref[...] = jnp.sum(log_p, dtype=jnp.float32)[None, None, None] * jnp.ones(
        (1, 1, 128), jnp.float32)


def _softlog(p):
    bits = lax.bitcast_convert_type(p, jnp.int32)
    e = jnp.right_shift(bits, 23) - 127
    m = lax.bitcast_convert_type((bits & 0x007FFFFF) | 0x3F800000, jnp.float32)
    big = m > _SQRT2
    m = jnp.where(big, m * np.float32(0.5), m)
    ef = (e + jnp.where(big, 1, 0)).astype(jnp.float32)
    f = m - np.float32(1.0)
    t = f / (np.float32(2.0) + f)
    t2 = t * t
    poly = np.float32(2.0) + t2 * (
        np.float32(2.0 / 3.0) + t2 * (
            np.float32(0.4) + t2 * (
                np.float32(2.0 / 7.0) + t2 * np.float32(2.0 / 9.0))))
    return t * poly + ef * _LN2


def _sc_body(pt, ot, hb, ceb, out_hbm, buf, hv, cev, outv):
    wid = lax.axis_index("s") * _NC + lax.axis_index("c")
    seg = _TCS + wid // _WPS
    part = wid % _WPS
    pltpu.sync_copy(hb.at[pl.ds(seg * _L, _L)], hv)
    pltpu.sync_copy(ceb.at[pl.ds(seg * _L, _L)], cev)
    hvec = hv[...]
    cevec = cev[...]
    omh = np.float32(1.0) - hvec
    row_base = seg * _ROWSEG + part * _WROWS

    def chunk_body(k, acc):
        row0 = row_base + k * _PR
        for c in range(3):
            pltpu.sync_copy(pt.at[pl.ds(c * _NROWS + row0, _PR)], buf.at[c])
            pltpu.sync_copy(ot.at[pl.ds(c * _NROWS + row0, _PR)], buf.at[3 + c])

        def inner(r, acc):
            for u in range(8):
                sl = pl.ds(u * _L, _L)
                dx = buf[0, r, sl] - buf[3, r, sl]
                dy = buf[1, r, sl] - buf[4, r, sl]
                dz = buf[2, r, sl] - buf[5, r, sl]
                s2 = dx * dx + dy * dy + dz * dz
                pe = hvec * jnp.exp(s2 * cevec) + omh
                pe = jnp.where(s2 < _T2, pe, np.float32(1.0))
                acc = acc + _softlog(pe)
            return acc

        return lax.fori_loop(0, _PR, inner, acc)

    acc = lax.fori_loop(0, _NCHUNK, chunk_body, jnp.zeros((_L,), jnp.float32))
    outv[...] = acc
    pltpu.sync_copy(outv, out_hbm.at[pl.ds(wid * _L, _L)])


@jax.jit
def kernel(u_pred, u_obs, h, lam):
    pt3 = u_pred.T.reshape(3, _R, _C)
    ot3 = u_obs.T.reshape(3, _R, _C)

    def scramble(x):
        return (x.reshape(3, _R // 8, 8, _C // 128, 128)
                 .transpose(0, 1, 3, 2, 4)
                 .reshape(3 * _NROWS, 128))

    ptv = scramble(pt3)
    otv = scramble(ot3)

    hb = jnp.broadcast_to(h[:, None], (_ELT, _L)).reshape(_ELT * _L)
    ceb = jnp.broadcast_to((lam * (-1.0 / _T2))[:, None],
                           (_ELT, _L)).reshape(_ELT * _L)

    mesh = plsc.VectorSubcoreMesh(core_axis_name="c", subcore_axis_name="s")
    out_sc = pl.kernel(
        _sc_body,
        mesh=mesh,
        out_type=jax.ShapeDtypeStruct((_NW * _L,), jnp.float32),
        scratch_types=[
            pltpu.VMEM((6, _PR, 128), jnp.float32),
            pltpu.VMEM((_L,), jnp.float32),
            pltpu.VMEM((_L,), jnp.float32),
            pltpu.VMEM((_L,), jnp.float32),
        ],
    )(ptv, otv, hb, ceb)

    hb3 = jnp.broadcast_to(h[:, None, None], (_ELT, 1, 128))
    lb3 = jnp.broadcast_to(lam[:, None, None], (_ELT, 1, 128))
    out_tc = pl.pallas_call(
        _tc_body,
        grid=(_TCS,),
        in_specs=[
            pl.BlockSpec((3, _RSEG, _C), lambda e: (0, e, 0)),
            pl.BlockSpec((3, _RSEG, _C), lambda e: (0, e, 0)),
            pl.BlockSpec((1, 1, 128), lambda e: (e, 0, 0)),
            pl.BlockSpec((1, 1, 128), lambda e: (e, 0, 0)),
        ],
        out_specs=pl.BlockSpec((1, 1, 128), lambda e: (e, 0, 0)),
        out_shape=jax.ShapeDtypeStruct((_TCS, 1, 128), jnp.float32),
    )(pt3, ot3, hb3, lb3)

    sc_sums = out_sc.reshape(_SCS, _WPS * _L).sum(axis=1)
    return jnp.concatenate([out_tc[:, 0, 0], sc_sums])


# speedup vs baseline: 2.6322x; 2.6322x over previous
import functools
import numpy as np
import jax
import jax.numpy as jnp
from jax.experimental import pallas as pl
from jax.experimental.pallas import tpu as pltpu

_ELT = 64
_ROW = 65536
_C = 1024
_R = _ELT * _ROW // _C
_RSEG = _ROW // _C
_SPB = 4
_T2 = np.float32((2.0 * np.sin(np.radians(10.0) / 2.0)) ** 2)


def _tc_body(p_ref, o_ref, h_ref, lam_ref, out_ref):
    d = p_ref[...] - o_ref[...]
    d2 = d * d
    s2 = d2[0] + d2[1] + d2[2]
    for i in range(_SPB):
        h = h_ref[i, 0, 0]
        lam = lam_ref[i, 0, 0]
        s2i = s2[i * _RSEG:(i + 1) * _RSEG]
        p = h * jnp.exp(s2i * (-1.0 / _T2) * lam) + (1.0 - h)
        log_p = jnp.where(s2i < _T2, jnp.log(p), 0.0)
        out_ref[i, :, :] = jnp.sum(log_p, dtype=jnp.float32)[None, None] * jnp.ones(
            (1, 128), jnp.float32)


@jax.jit
def kernel(u_pred, u_obs, h, lam):
    pt = u_pred.T.reshape(3, _R, _C)
    ot = u_obs.T.reshape(3, _R, _C)
    hb = jnp.broadcast_to(h[:, None, None], (_ELT, 1, 128))
    lb = jnp.broadcast_to(lam[:, None, None], (_ELT, 1, 128))
    out = pl.pallas_call(
        _tc_body,
        grid=(_ELT // _SPB,),
        in_specs=[
            pl.BlockSpec((3, _SPB * _RSEG, _C), lambda e: (0, e, 0)),
            pl.BlockSpec((3, _SPB * _RSEG, _C), lambda e: (0, e, 0)),
            pl.BlockSpec((_SPB, 1, 128), lambda e: (e, 0, 0)),
            pl.BlockSpec((_SPB, 1, 128), lambda e: (e, 0, 0)),
        ],
        out_specs=pl.BlockSpec((_SPB, 1, 128), lambda e: (e, 0, 0)),
        out_shape=jax.ShapeDtypeStruct((_ELT, 1, 128), jnp.float32),
    )(pt, ot, hb, lb)
    return out[:, 0, 0]


# speedup vs baseline: 2.6655x; 1.0126x over previous
import functools
import numpy as np
import jax
import jax.numpy as jnp
from jax.experimental import pallas as pl
from jax.experimental.pallas import tpu as pltpu

_ELT = 64
_ROW = 65536
_C = 1024
_R = _ELT * _ROW // _C
_RSEG = _ROW // _C
_SPB = 8
_T2 = np.float32((2.0 * np.sin(np.radians(10.0) / 2.0)) ** 2)


def _tc_body(p_ref, o_ref, h_ref, lam_ref, out_ref):
    d = p_ref[...] - o_ref[...]
    d2 = d * d
    s2 = d2[0] + d2[1] + d2[2]
    for i in range(_SPB):
        h = h_ref[i, 0, 0]
        lam = lam_ref[i, 0, 0]
        s2i = s2[i * _RSEG:(i + 1) * _RSEG]
        p = h * jnp.exp(s2i * (-1.0 / _T2) * lam) + (1.0 - h)
        log_p = jnp.where(s2i < _T2, jnp.log(p), 0.0)
        out_ref[i, :, :] = jnp.sum(log_p, dtype=jnp.float32)[None, None] * jnp.ones(
            (1, 128), jnp.float32)


@jax.jit
def kernel(u_pred, u_obs, h, lam):
    pt = u_pred.T.reshape(3, _R, _C)
    ot = u_obs.T.reshape(3, _R, _C)
    hb = jnp.broadcast_to(h[:, None, None], (_ELT, 1, 128))
    lb = jnp.broadcast_to(lam[:, None, None], (_ELT, 1, 128))
    out = pl.pallas_call(
        _tc_body,
        grid=(_ELT // _SPB,),
        in_specs=[
            pl.BlockSpec((3, _SPB * _RSEG, _C), lambda e: (0, e, 0)),
            pl.BlockSpec((3, _SPB * _RSEG, _C), lambda e: (0, e, 0)),
            pl.BlockSpec((_SPB, 1, 128), lambda e: (e, 0, 0)),
            pl.BlockSpec((_SPB, 1, 128), lambda e: (e, 0, 0)),
        ],
        out_specs=pl.BlockSpec((_SPB, 1, 128), lambda e: (e, 0, 0)),
        out_shape=jax.ShapeDtypeStruct((_ELT, 1, 128), jnp.float32),
    )(pt, ot, hb, lb)
    return out[:, 0, 0]
